# baseline (speedup 1.0000x reference)
import functools

import jax
import jax.numpy as jnp
import numpy as np
from jax import lax
from jax.experimental import pallas as pl
from jax.experimental.pallas import tpu as pltpu
from jax.experimental.pallas import tpu_sc as plsc

_M = 4096
_K = 4096
_NNZ_PER_ROW = 409
_NNZ = _M * _NNZ_PER_ROW
_NCOL = 256

_NW = 32
_ROWS_PER_W = _M // _NW
_G = 16
_NG = _ROWS_PER_W // _G
_REG = _G * _K
_NNZ_G = _G * _NNZ_PER_ROW
_CH = 128
_NCH = (_NNZ_G + _CH - 1) // _CH
_PAD_G = _NCH * _CH
_ZW = 16384

_MH = _M // 2
_RPW_H = _MH // _NW
_NG_H = _RPW_H // _G
_NNZ_W_H = _RPW_H * _NNZ_PER_ROW
_NNZ_H = _MH * _NNZ_PER_ROW

_GRP_BASE = ((np.arange(_NNZ_G, dtype=np.int64) // _NNZ_PER_ROW)
             * _K).astype(np.int32)
_BASE_PAD = np.full((_NW, _NG, _PAD_G), 16 * _REG, dtype=np.int32)
_BASE_PAD[:, :, :_NNZ_G] = (_GRP_BASE[None, None, :]
                            + (np.arange(_NW, dtype=np.int32) // 2
                               * _REG)[:, None, None])

_TM = 256


def _densify(vals_flat, idx3, half):
    mesh = plsc.VectorSubcoreMesh(core_axis_name="c", subcore_axis_name="s")

    @functools.partial(
        pl.kernel,
        out_type=jax.ShapeDtypeStruct((_MH, _K), jnp.float32),
        mesh=mesh,
        scratch_types=[
            pltpu.VMEM_SHARED((16 * _REG + 8,), jnp.float32),
            pltpu.VMEM((_PAD_G,), jnp.int32),
            pltpu.VMEM((_PAD_G,), jnp.float32),
            pltpu.VMEM((_ZW,), jnp.float32),
            pltpu.SemaphoreType.DMA,
            pltpu.SemaphoreType.DMA,
            pltpu.SemaphoreType.DMA,
        ],
    )
    def k(zeros_hbm, vals_hbm, idx_hbm, w_hbm, shared, idx_v, vals_v, zero_v,
          zsem, ssem, osem):
        c = lax.axis_index("c")
        s = lax.axis_index("s")
        wid = s * 2 + c
        base = s * _REG
        row_base = wid * _RPW_H
        nnz_base = wid * (_NG * _NNZ_G) + half * (_NG_H * _NNZ_G)
        g_base = half * _NG_H

        pltpu.sync_copy(zeros_hbm, zero_v)

        def drain_out(g):
            def d(i, carry):
                pltpu.make_async_copy(shared.at[pl.ds(base + i * _K, _K)],
                                      w_hbm.at[row_base + g * _G + i],
                                      osem).wait()
                return carry
            lax.fori_loop(0, _G, d, 0)

        def group(g, carry):
            @pl.when(g > 0)
            def _():
                drain_out(g - 1)

            def zf(z, carry2):
                pltpu.async_copy(zero_v, shared.at[pl.ds(base + z * _ZW, _ZW)],
                                 zsem)
                return carry2

            lax.fori_loop(0, _REG // _ZW, zf, 0)
            pltpu.sync_copy(idx_hbm.at[wid, g_base + g], idx_v)
            pltpu.sync_copy(vals_hbm.at[pl.ds(nnz_base + g * _NNZ_G, _PAD_G)],
                            vals_v)

            def zd(z, carry2):
                pltpu.make_async_copy(zero_v,
                                      shared.at[pl.ds(base + z * _ZW, _ZW)],
                                      zsem).wait()
                return carry2

            lax.fori_loop(0, _REG // _ZW, zd, 0)

            pltpu.async_copy(vals_v, shared.at[idx_v], ssem, add=True)
            pltpu.make_async_copy(vals_v, shared.at[idx_v], ssem).wait()

            def of(i, carry2):
                pltpu.async_copy(shared.at[pl.ds(base + i * _K, _K)],
                                 w_hbm.at[row_base + g * _G + i], osem)
                return carry2

            lax.fori_loop(0, _G, of, 0)
            return carry

        lax.fori_loop(0, _NG_H, group, 0)
        drain_out(_NG_H - 1)

    return k(jnp.zeros((_ZW,), jnp.float32), vals_flat, idx3)


def _mm_body(w_ref, x_ref, b_ref, o_ref):
    o_ref[...] = jnp.dot(w_ref[...], x_ref[...],
                         preferred_element_type=jnp.float32) + b_ref[0:1, :]


def _matmul(w, x2, bias_flat):
    grid = (_MH // _TM,)
    return pl.pallas_call(
        _mm_body,
        grid=grid,
        in_specs=[
            pl.BlockSpec((_TM, _K), lambda m: (m, 0)),
            pl.BlockSpec((_K, _NCOL), lambda m: (0, 0)),
            pl.BlockSpec((8, _NCOL), lambda m: (0, 0)),
        ],
        out_specs=pl.BlockSpec((_TM, _NCOL), lambda m: (m, 0)),
        out_shape=jax.ShapeDtypeStruct((_MH, _NCOL), jnp.float32),
    )(w, x2, bias_flat)


def kernel(x, values, row_indices, row_offsets, column_indices, bias):
    B, SEQ, K = x.shape
    cols_pad = jnp.pad(column_indices.reshape(_NW, _NG, _NNZ_G),
                       ((0, 0), (0, 0), (0, _PAD_G - _NNZ_G)))
    idx3 = jnp.asarray(_BASE_PAD) + cols_pad
    vals_flat = jnp.pad(values, (0, _CH))

    x2 = jnp.transpose(x, (2, 0, 1)).reshape(K, B * SEQ)
    bias_flat = jnp.broadcast_to(jnp.tile(bias, B)[None, :], (8, B * SEQ))

    W0 = _densify(vals_flat, idx3, 0)
    W1 = _densify(vals_flat, idx3, 1)
    o0 = _matmul(W0, x2, bias_flat)
    o1 = _matmul(W1, x2, bias_flat)
    of = jnp.stack([o0.reshape(_NW, _RPW_H, B * SEQ),
                    o1.reshape(_NW, _RPW_H, B * SEQ)], axis=1)
    return jnp.transpose(of.reshape(_M, B, SEQ), (1, 0, 2))


# speedup vs baseline: 1.0967x; 1.0967x over previous
import functools

import jax
import jax.numpy as jnp
import numpy as np
from jax import lax
from jax.experimental import pallas as pl
from jax.experimental.pallas import tpu as pltpu
from jax.experimental.pallas import tpu_sc as plsc

_M = 4096
_K = 4096
_NNZ_PER_ROW = 409
_NNZ = _M * _NNZ_PER_ROW
_NCOL = 256

_NW = 32
_ROWS_PER_W = _M // _NW
_G = 16
_NG = _ROWS_PER_W // _G
_REG = _G * _K
_NNZ_G = _G * _NNZ_PER_ROW
_CH = 128
_NCH = (_NNZ_G + _CH - 1) // _CH
_PAD_G = _NCH * _CH
_ZW = 16384

_GRP_BASE = ((np.arange(_NNZ_G, dtype=np.int64) // _NNZ_PER_ROW)
             * _K).astype(np.int32)
_BASE_PAD = np.full((_NW, _NG, _PAD_G), 16 * _REG, dtype=np.int32)
_BASE_PAD[:, :, :_NNZ_G] = (_GRP_BASE[None, None, :]
                            + (np.arange(_NW, dtype=np.int32) // 2
                               * _REG)[:, None, None])

_TM = 256


def _densify(vals_flat, idx3):
    mesh = plsc.VectorSubcoreMesh(core_axis_name="c", subcore_axis_name="s")

    @functools.partial(
        pl.kernel,
        out_type=jax.ShapeDtypeStruct((_M, _K), jnp.float32),
        mesh=mesh,
        scratch_types=[
            pltpu.VMEM_SHARED((16 * _REG + 8,), jnp.float32),
            pltpu.VMEM((_PAD_G,), jnp.int32),
            pltpu.VMEM((_PAD_G,), jnp.float32),
            pltpu.VMEM((_ZW,), jnp.float32),
            pltpu.SemaphoreType.DMA,
            pltpu.SemaphoreType.DMA,
            pltpu.SemaphoreType.DMA,
        ],
    )
    def k(zeros_hbm, vals_hbm, idx_hbm, w_hbm, shared, idx_v, vals_v, zero_v,
          zsem, ssem, osem):
        c = lax.axis_index("c")
        s = lax.axis_index("s")
        wid = s * 2 + c
        base = s * _REG
        row_base = wid * _ROWS_PER_W
        nnz_base = wid * (_NG * _NNZ_G)

        pltpu.sync_copy(zeros_hbm, zero_v)

        def drain_out(g):
            def d(i, carry):
                pltpu.make_async_copy(shared.at[pl.ds(base + i * _K, _K)],
                                      w_hbm.at[row_base + g * _G + i],
                                      osem).wait()
                return carry
            lax.fori_loop(0, _G, d, 0)

        def group(g, carry):
            @pl.when(g > 0)
            def _():
                drain_out(g - 1)

            def zf(z, carry2):
                pltpu.async_copy(zero_v, shared.at[pl.ds(base + z * _ZW, _ZW)],
                                 zsem)
                return carry2

            lax.fori_loop(0, _REG // _ZW, zf, 0)
            pltpu.sync_copy(idx_hbm.at[wid, g], idx_v)
            pltpu.sync_copy(vals_hbm.at[pl.ds(nnz_base + g * _NNZ_G, _PAD_G)],
                            vals_v)

            def zd(z, carry2):
                pltpu.make_async_copy(zero_v,
                                      shared.at[pl.ds(base + z * _ZW, _ZW)],
                                      zsem).wait()
                return carry2

            lax.fori_loop(0, _REG // _ZW, zd, 0)

            pltpu.async_copy(vals_v, shared.at[idx_v], ssem, add=True)
            pltpu.make_async_copy(vals_v, shared.at[idx_v], ssem).wait()

            def of(i, carry2):
                pltpu.async_copy(shared.at[pl.ds(base + i * _K, _K)],
                                 w_hbm.at[row_base + g * _G + i], osem)
                return carry2

            lax.fori_loop(0, _G, of, 0)
            return carry

        lax.fori_loop(0, _NG, group, 0)
        drain_out(_NG - 1)

    return k(jnp.zeros((_ZW,), jnp.float32), vals_flat, idx3)


def _mm_body(w_ref, x_ref, b_ref, o_ref):
    o_ref[...] = jnp.dot(w_ref[...], x_ref[...],
                         preferred_element_type=jnp.float32) + b_ref[0:1, :]


def _matmul(w, x2, bias_flat):
    grid = (_M // _TM,)
    return pl.pallas_call(
        _mm_body,
        grid=grid,
        in_specs=[
            pl.BlockSpec((_TM, _K), lambda m: (m, 0)),
            pl.BlockSpec((_K, _NCOL), lambda m: (0, 0)),
            pl.BlockSpec((8, _NCOL), lambda m: (0, 0)),
        ],
        out_specs=pl.BlockSpec((_TM, _NCOL), lambda m: (m, 0)),
        out_shape=jax.ShapeDtypeStruct((_M, _NCOL), jnp.float32),
    )(w, x2, bias_flat)


def kernel(x, values, row_indices, row_offsets, column_indices, bias):
    B, SEQ, K = x.shape
    cols_pad = jnp.pad(column_indices.reshape(_NW, _NG, _NNZ_G),
                       ((0, 0), (0, 0), (0, _PAD_G - _NNZ_G)))
    idx3 = jnp.asarray(_BASE_PAD) + cols_pad
    vals_flat = jnp.pad(values, (0, _CH))

    x2 = jnp.transpose(x, (2, 0, 1)).reshape(K, B * SEQ)
    bias_flat = jnp.broadcast_to(jnp.tile(bias, B)[None, :], (8, B * SEQ))

    W = _densify(vals_flat, idx3)
    out_flat = _matmul(W, x2, bias_flat)
    return jnp.transpose(out_flat.reshape(_M, B, SEQ), (1, 0, 2))
